# baseline (speedup 1.0000x reference)
import jax
import jax.numpy as jnp
from jax.experimental import pallas as pl
from jax.experimental.pallas import tpu as pltpu

B = 128
DIM = 32768
HID = 64
STEP = 0.4
BAL = 1.0
TEMP = 1.0
EPS = 1e-10
TERM2 = 1.0 / (2.0 * STEP)

C = 2048
N = DIM // C

_HI = jax.lax.Precision.HIGHEST


def _dot(a, b):
    return jax.lax.dot_general(a, b, (((1,), (0,)), ((), ())),
                               precision=_HI, preferred_element_type=jnp.float32)


def _stage1(x_j, W_j, b_j, h_ref, xb_ref):
    j = pl.program_id(0)

    @pl.when(j == 0)
    def _():
        h_ref[...] = jnp.zeros_like(h_ref)
        xb_ref[...] = jnp.zeros_like(xb_ref)

    x = x_j[...]
    h_ref[...] += _dot(x, W_j[...])
    xb_ref[...] += jnp.sum(x * b_j[...], axis=1, keepdims=True)


def _stage2(x_j, u_j, Wt_j, W_j, b_j, h, xd_ref, lpf_ref, hd_ref, xdb_ref):
    j = pl.program_id(0)

    @pl.when(j == 0)
    def _():
        lpf_ref[...] = jnp.zeros_like(lpf_ref)
        hd_ref[...] = jnp.zeros_like(hd_ref)
        xdb_ref[...] = jnp.zeros_like(xdb_ref)

    x = x_j[...]
    grad = b_j[...] - _dot(h[...], Wt_j[...])
    z = BAL * (1.0 - 2.0 * x) * grad - TERM2
    fp = jax.nn.sigmoid(z)
    ind = (u_j[...] < fp).astype(jnp.float32)
    xd = x + ind - 2.0 * x * ind
    xd_ref[...] = xd
    probs = fp * ind + (1.0 - fp) * (1.0 - ind)
    lpf_ref[...] += jnp.sum(jnp.log(probs + EPS), axis=1, keepdims=True)
    hd_ref[...] += _dot(xd, W_j[...])
    xdb_ref[...] += jnp.sum(xd * b_j[...], axis=1, keepdims=True)


def _stage3(x_j, xd_j, Wt_j, b_j, u2, h, hd, xb, xdb, lpf, a_ref, lpr_ref):
    j = pl.program_id(0)

    @pl.when(j == 0)
    def _():
        lpr_ref[...] = jnp.zeros_like(lpr_ref)

    x = x_j[...]
    xd = xd_j[...]
    ind = jnp.abs(xd - x)
    grad_d = b_j[...] - _dot(hd[...], Wt_j[...])
    zr = BAL * (1.0 - 2.0 * xd) * grad_d - TERM2
    fpr = jax.nn.sigmoid(zr)
    probs_r = fpr * ind + (1.0 - fpr) * (1.0 - ind)
    lpr_ref[...] += jnp.sum(jnp.log(probs_r + EPS), axis=1, keepdims=True)

    @pl.when(j == N - 1)
    def _():
        hh = h[...]
        hdd = hd[...]
        m = (xdb[...] - 0.5 * jnp.sum(hdd * hdd, axis=1, keepdims=True)) \
            - (xb[...] - 0.5 * jnp.sum(hh * hh, axis=1, keepdims=True))
        la = m * TEMP + lpr_ref[...] - lpf[...]
        a_ref[...] = (jnp.log(u2[...] + EPS) < la).astype(jnp.float32)


def _stage4(x_j, xd_j, a, out_j):
    av = a[...]
    out_j[...] = xd_j[...] * av + x_j[...] * (1.0 - av)


def kernel(x, W, b, u, u2):
    Wt = W.T
    b2 = b.reshape(1, DIM)
    u2c = u2.reshape(B, 1)
    f32 = jnp.float32

    blk_x = pl.BlockSpec((B, C), lambda j: (0, j))
    blk_W = pl.BlockSpec((C, HID), lambda j: (j, 0))
    blk_Wt = pl.BlockSpec((HID, C), lambda j: (0, j))
    blk_b = pl.BlockSpec((1, C), lambda j: (0, j))
    full = lambda shape: pl.BlockSpec(shape, lambda j: (0, 0))

    h, xb = pl.pallas_call(
        _stage1,
        grid=(N,),
        in_specs=[blk_x, blk_W, blk_b],
        out_specs=[full((B, HID)), full((B, 1))],
        out_shape=[jax.ShapeDtypeStruct((B, HID), f32),
                   jax.ShapeDtypeStruct((B, 1), f32)],
    )(x, W, b2)

    xd, lpf, hd, xdb = pl.pallas_call(
        _stage2,
        grid=(N,),
        in_specs=[blk_x, blk_x, blk_Wt, blk_W, blk_b, full((B, HID))],
        out_specs=[blk_x, full((B, 1)), full((B, HID)), full((B, 1))],
        out_shape=[jax.ShapeDtypeStruct((B, DIM), f32),
                   jax.ShapeDtypeStruct((B, 1), f32),
                   jax.ShapeDtypeStruct((B, HID), f32),
                   jax.ShapeDtypeStruct((B, 1), f32)],
    )(x, u, Wt, W, b2, h)

    a = pl.pallas_call(
        _stage3,
        grid=(N,),
        in_specs=[blk_x, blk_x, blk_Wt, blk_b, full((B, 1)),
                  full((B, HID)), full((B, HID)), full((B, 1)),
                  full((B, 1)), full((B, 1))],
        out_specs=full((B, 1)),
        out_shape=jax.ShapeDtypeStruct((B, 1), f32),
        scratch_shapes=[pltpu.VMEM((B, 1), f32)],
    )(x, xd, Wt, b2, u2c, h, hd, xb, xdb, lpf)

    x_new = pl.pallas_call(
        _stage4,
        grid=(N,),
        in_specs=[blk_x, blk_x, full((B, 1))],
        out_specs=blk_x,
        out_shape=jax.ShapeDtypeStruct((B, DIM), f32),
    )(x, xd, a)

    return x_new


# speedup vs baseline: 1.3372x; 1.3372x over previous
import jax
import jax.numpy as jnp
from jax.experimental import pallas as pl
from jax.experimental.pallas import tpu as pltpu

B = 128
DIM = 32768
HID = 64
STEP = 0.4
BAL = 1.0
TEMP = 1.0
EPS = 1e-10
TERM2 = 1.0 / (2.0 * STEP)

C = 2048
N = DIM // C

bf16 = jnp.bfloat16
f32 = jnp.float32


def _dot(a, b):
    return jax.lax.dot_general(a, b, (((1,), (0,)), ((), ())),
                               preferred_element_type=f32)


def _split(v):
    hi = v.astype(bf16)
    lo = (v - hi.astype(f32)).astype(bf16)
    return hi, lo


def _body(x_j, u_j, whi_j, wlo_j, wthi_j, wtlo_j, b_j, u2,
          out_j,
          xc, xdc, whc, wlc, wthc, wtlc,
          h_ref, hd_ref, xb_ref, xdb_ref, lpf_ref, lpr_ref, a_ref):
    p = pl.program_id(0)
    j = pl.program_id(1)
    cols = pl.ds(j * C, C)

    @pl.when(p == 0)
    def _phase0():
        @pl.when(j == 0)
        def _():
            h_ref[...] = jnp.zeros_like(h_ref)
            xb_ref[...] = jnp.zeros_like(xb_ref)

        x = x_j[...]
        x16 = x.astype(bf16)
        xc[:, cols] = x16
        whi = whi_j[...]
        wlo = wlo_j[...]
        whc[cols, :] = whi
        wlc[cols, :] = wlo
        h_ref[...] += _dot(x16, whi) + _dot(x16, wlo)
        xb_ref[...] += jnp.sum(x * b_j[...], axis=1, keepdims=True)

    @pl.when(p == 1)
    def _phase1():
        @pl.when(j == 0)
        def _():
            hd_ref[...] = jnp.zeros_like(hd_ref)
            xdb_ref[...] = jnp.zeros_like(xdb_ref)
            lpf_ref[...] = jnp.zeros_like(lpf_ref)

        wthi = wthi_j[...]
        wtlo = wtlo_j[...]
        wthc[:, cols] = wthi
        wtlc[:, cols] = wtlo
        h_hi, h_lo = _split(h_ref[...])
        grad = b_j[...] - (_dot(h_hi, wthi) + _dot(h_hi, wtlo)
                           + _dot(h_lo, wthi))
        x = xc[:, cols].astype(f32)
        z = BAL * (1.0 - 2.0 * x) * grad - TERM2
        fp = jax.nn.sigmoid(z)
        ind = u_j[...] < fp
        xd = jnp.where(ind, 1.0 - x, x)
        xd16 = xd.astype(bf16)
        xdc[:, cols] = xd16
        probs = jnp.where(ind, fp, 1.0 - fp)
        lpf_ref[...] += jnp.sum(jnp.log(probs + EPS), axis=1, keepdims=True)
        hd_ref[...] += _dot(xd16, whc[cols, :]) + _dot(xd16, wlc[cols, :])
        xdb_ref[...] += jnp.sum(xd * b_j[...], axis=1, keepdims=True)

    @pl.when(p == 2)
    def _phase2():
        @pl.when(j == 0)
        def _():
            lpr_ref[...] = jnp.zeros_like(lpr_ref)

        hd_hi, hd_lo = _split(hd_ref[...])
        grad_d = b_j[...] - (_dot(hd_hi, wthc[:, cols]) + _dot(hd_hi, wtlc[:, cols])
                             + _dot(hd_lo, wthc[:, cols]))
        x = xc[:, cols].astype(f32)
        xd = xdc[:, cols].astype(f32)
        ind = jnp.abs(xd - x) > 0.5
        zr = BAL * (1.0 - 2.0 * xd) * grad_d - TERM2
        fpr = jax.nn.sigmoid(zr)
        probs_r = jnp.where(ind, fpr, 1.0 - fpr)
        lpr_ref[...] += jnp.sum(jnp.log(probs_r + EPS), axis=1, keepdims=True)

        @pl.when(j == N - 1)
        def _():
            h = h_ref[...]
            hd = hd_ref[...]
            m = (xdb_ref[...] - 0.5 * jnp.sum(hd * hd, axis=1, keepdims=True)) \
                - (xb_ref[...] - 0.5 * jnp.sum(h * h, axis=1, keepdims=True))
            la = m * TEMP + lpr_ref[...] - lpf_ref[...]
            a_ref[...] = (jnp.log(u2[...] + EPS) < la).astype(f32)

    @pl.when(p == 3)
    def _phase3():
        x = xc[:, cols].astype(f32)
        xd = xdc[:, cols].astype(f32)
        out_j[...] = jnp.where(a_ref[...] > 0.5, xd, x)


def kernel(x, W, b, u, u2):
    W_hi = W.astype(bf16)
    W_lo = (W - W_hi.astype(f32)).astype(bf16)
    Wt_hi = W_hi.T
    Wt_lo = W_lo.T
    b2 = b.reshape(1, DIM)
    u2c = u2.reshape(B, 1)

    blk_x = pl.BlockSpec((B, C), lambda p, j: (0, jnp.where(p == 0, j, 0)))
    blk_u = pl.BlockSpec((B, C), lambda p, j: (0, jnp.where(p == 1, j, 0)))
    blk_W = pl.BlockSpec((C, HID), lambda p, j: (jnp.where(p == 0, j, 0), 0))
    blk_Wt = pl.BlockSpec((HID, C), lambda p, j: (0, jnp.where(p == 1, j, 0)))
    blk_b = pl.BlockSpec((1, C), lambda p, j: (0, jnp.where(p < 3, j, 0)))
    blk_u2 = pl.BlockSpec((B, 1), lambda p, j: (0, 0))
    blk_out = pl.BlockSpec((B, C), lambda p, j: (0, jnp.where(p == 3, j, 0)))

    return pl.pallas_call(
        _body,
        grid=(4, N),
        in_specs=[blk_x, blk_u, blk_W, blk_W, blk_Wt, blk_Wt, blk_b, blk_u2],
        out_specs=blk_out,
        out_shape=jax.ShapeDtypeStruct((B, DIM), f32),
        scratch_shapes=[
            pltpu.VMEM((B, DIM), bf16),
            pltpu.VMEM((B, DIM), bf16),
            pltpu.VMEM((DIM, HID), bf16),
            pltpu.VMEM((DIM, HID), bf16),
            pltpu.VMEM((HID, DIM), bf16),
            pltpu.VMEM((HID, DIM), bf16),
            pltpu.VMEM((B, HID), f32),
            pltpu.VMEM((B, HID), f32),
            pltpu.VMEM((B, 1), f32),
            pltpu.VMEM((B, 1), f32),
            pltpu.VMEM((B, 1), f32),
            pltpu.VMEM((B, 1), f32),
            pltpu.VMEM((B, 1), f32),
        ],
    )(x, u, W_hi, W_lo, Wt_hi, Wt_lo, b2, u2c)


# speedup vs baseline: 1.3525x; 1.0115x over previous
import jax
import jax.numpy as jnp
from jax.experimental import pallas as pl
from jax.experimental.pallas import tpu as pltpu

B = 128
DIM = 32768
HID = 64
STEP = 0.4
BAL = 1.0
TEMP = 1.0
EPS = 1e-10
TERM2 = 1.0 / (2.0 * STEP)

C = 4096
N = DIM // C

bf16 = jnp.bfloat16
f32 = jnp.float32


def _dot(a, b):
    return jax.lax.dot_general(a, b, (((1,), (0,)), ((), ())),
                               preferred_element_type=f32)


def _acc_chunks(acc_ref, vals):
    s = vals[:, 0:128]
    for k in range(1, C // 128):
        s = s + vals[:, k * 128:(k + 1) * 128]
    acc_ref[...] += s


def _split_cat3(v):
    hi = v.astype(bf16)
    lo = (v - hi.astype(f32)).astype(bf16)
    return jnp.concatenate([hi, hi, lo], axis=1)


def _body(x_j, u_j, wcat_j, wtcat_j, b_j, u2,
          out_j,
          xc, xdc, wcat_c, wtcat_c,
          hv_ref, hdv_ref, hcat_ref, hdcat_ref,
          xbv, xdbv, lpfv, lprv, a_ref):
    p = pl.program_id(0)
    j = pl.program_id(1)
    cols = pl.ds(j * C, C)

    @pl.when(p == 0)
    def _phase0():
        @pl.when(j == 0)
        def _():
            hv_ref[...] = jnp.zeros_like(hv_ref)
            xbv[...] = jnp.zeros_like(xbv)

        x = x_j[...]
        x16 = x.astype(bf16)
        xc[:, cols] = x16
        wcat = wcat_j[...]
        wcat_c[cols, :] = wcat
        hv_ref[...] += _dot(x16, wcat)
        _acc_chunks(xbv, x * b_j[...])

    @pl.when(p == 1)
    def _phase1():
        @pl.when(j == 0)
        def _():
            hdv_ref[...] = jnp.zeros_like(hdv_ref)
            xdbv[...] = jnp.zeros_like(xdbv)
            lpfv[...] = jnp.zeros_like(lpfv)
            hv = hv_ref[...]
            hcat_ref[...] = _split_cat3(hv[:, 0:HID] + hv[:, HID:2 * HID])

        wtcat = wtcat_j[...]
        wtcat_c[:, cols] = wtcat
        grad = b_j[...] - _dot(hcat_ref[...], wtcat)
        x = xc[:, cols].astype(f32)
        z = BAL * (1.0 - 2.0 * x) * grad - TERM2
        w = jnp.exp(-z)
        ind = u_j[...] * (1.0 + w) < 1.0
        xd = jnp.where(ind, 1.0 - x, x)
        xdc[:, cols] = xd.astype(bf16)
        lw = jnp.log1p(w)
        _acc_chunks(lpfv, jnp.where(ind, -lw, -(z + lw)))
        hdv_ref[...] += _dot(xd.astype(bf16), wcat_c[cols, :])
        _acc_chunks(xdbv, xd * b_j[...])

    @pl.when(p == 2)
    def _phase2():
        @pl.when(j == 0)
        def _():
            lprv[...] = jnp.zeros_like(lprv)
            hdv = hdv_ref[...]
            hdcat_ref[...] = _split_cat3(hdv[:, 0:HID] + hdv[:, HID:2 * HID])

        grad_d = b_j[...] - _dot(hdcat_ref[...], wtcat_c[:, cols])
        x = xc[:, cols].astype(f32)
        xd = xdc[:, cols].astype(f32)
        ind = jnp.abs(xd - x) > 0.5
        zr = BAL * (1.0 - 2.0 * xd) * grad_d - TERM2
        wr = jnp.exp(-zr)
        lwr = jnp.log1p(wr)
        _acc_chunks(lprv, jnp.where(ind, -lwr, -(zr + lwr)))

        @pl.when(j == N - 1)
        def _():
            hv = hv_ref[...]
            h = hv[:, 0:HID] + hv[:, HID:2 * HID]
            hdv = hdv_ref[...]
            hd = hdv[:, 0:HID] + hdv[:, HID:2 * HID]
            xb = jnp.sum(xbv[...], axis=1, keepdims=True)
            xdb = jnp.sum(xdbv[...], axis=1, keepdims=True)
            lpf = jnp.sum(lpfv[...], axis=1, keepdims=True)
            lpr = jnp.sum(lprv[...], axis=1, keepdims=True)
            m = (xdb - 0.5 * jnp.sum(hd * hd, axis=1, keepdims=True)) \
                - (xb - 0.5 * jnp.sum(h * h, axis=1, keepdims=True))
            la = m * TEMP + lpr - lpf
            a_ref[...] = (jnp.log(u2[...] + EPS) < la).astype(f32)

    @pl.when(p == 3)
    def _phase3():
        x = xc[:, cols].astype(f32)
        xd = xdc[:, cols].astype(f32)
        out_j[...] = jnp.where(a_ref[...] > 0.5, xd, x)


def kernel(x, W, b, u, u2):
    W_hi = W.astype(bf16)
    W_lo = (W - W_hi.astype(f32)).astype(bf16)
    Wcat = jnp.concatenate([W_hi, W_lo], axis=1)
    Wtcat = jnp.concatenate([W_hi.T, W_lo.T, W_hi.T], axis=0)
    b2 = b.reshape(1, DIM)
    u2c = u2.reshape(B, 1)

    blk_x = pl.BlockSpec((B, C), lambda p, j: (0, jnp.where(p == 0, j, 0)))
    blk_u = pl.BlockSpec((B, C), lambda p, j: (0, jnp.where(p == 1, j, 0)))
    blk_W = pl.BlockSpec((C, 2 * HID), lambda p, j: (jnp.where(p == 0, j, 0), 0))
    blk_Wt = pl.BlockSpec((3 * HID, C), lambda p, j: (0, jnp.where(p == 1, j, 0)))
    blk_b = pl.BlockSpec((1, C), lambda p, j: (0, jnp.where(p < 3, j, 0)))
    blk_u2 = pl.BlockSpec((B, 1), lambda p, j: (0, 0))
    blk_out = pl.BlockSpec((B, C), lambda p, j: (0, jnp.where(p == 3, j, 0)))

    return pl.pallas_call(
        _body,
        grid=(4, N),
        in_specs=[blk_x, blk_u, blk_W, blk_Wt, blk_b, blk_u2],
        out_specs=blk_out,
        out_shape=jax.ShapeDtypeStruct((B, DIM), f32),
        scratch_shapes=[
            pltpu.VMEM((B, DIM), bf16),
            pltpu.VMEM((B, DIM), bf16),
            pltpu.VMEM((DIM, 2 * HID), bf16),
            pltpu.VMEM((3 * HID, DIM), bf16),
            pltpu.VMEM((B, 2 * HID), f32),
            pltpu.VMEM((B, 2 * HID), f32),
            pltpu.VMEM((B, 3 * HID), bf16),
            pltpu.VMEM((B, 3 * HID), bf16),
            pltpu.VMEM((B, 128), f32),
            pltpu.VMEM((B, 128), f32),
            pltpu.VMEM((B, 128), f32),
            pltpu.VMEM((B, 128), f32),
            pltpu.VMEM((B, 1), f32),
        ],
    )(x, u, Wcat, Wtcat, b2, u2c)
